# Initial kernel scaffold; baseline (speedup 1.0000x reference)
import jax
import jax.numpy as jnp
from jax.experimental import pallas as pl


def kernel(words_tensor, words_lengths, doc_tensor, doc_tensor_new_dx, char_table, word_table, W_ih, W_hh, b_ih, b_hh):
    raise NotImplementedError("write your pallas kernel here")


# speedup vs baseline: 4.1581x; 4.1581x over previous
import functools

import jax
import jax.numpy as jnp
from jax import lax
from jax.experimental import pallas as pl
from jax.experimental.pallas import tpu as pltpu
from jax.experimental.pallas import tpu_sc as plsc

CHAR_VOCAB = 128
CHAR_DIM = 16
HID = 64
WORD_DIM = 64
NW = 8192
TW = 16
B = 1024
L = 50
TOT = B * L

NB = 2048

N_WORKERS = 32
PER_W = TOT // N_WORKERS
CHUNK = 100
N_CHUNKS = PER_W // CHUNK


def _gru_block(chars_ref, lens_ref, ct_ref,
               wxr_ref, wxz_ref, wxn_ref, whr_ref, whz_ref, whn_ref,
               bxr_ref, bxz_ref, bxn_ref, bhr_ref, bhz_ref, bhn_ref,
               h_ref):
    f32 = jnp.float32
    ct = ct_ref[...]
    ct = jnp.where(lax.broadcasted_iota(jnp.int32, ct.shape, 0) == 0, 0.0, ct)
    chars = chars_ref[...]
    lens = lens_ref[...]
    iota_v = lax.broadcasted_iota(jnp.int32, (1, CHAR_VOCAB), 1)
    h = jnp.zeros((chars.shape[0], HID), f32)
    dot = functools.partial(jnp.dot, preferred_element_type=f32)
    for t in range(TW):
        oh = (chars[:, t:t + 1] == iota_v).astype(f32)
        x = dot(oh, ct)
        r = jax.nn.sigmoid(dot(x, wxr_ref[...]) + bxr_ref[...]
                           + dot(h, whr_ref[...]) + bhr_ref[...])
        z = jax.nn.sigmoid(dot(x, wxz_ref[...]) + bxz_ref[...]
                           + dot(h, whz_ref[...]) + bhz_ref[...])
        n = jnp.tanh(dot(x, wxn_ref[...]) + bxn_ref[...]
                     + r * (dot(h, whn_ref[...]) + bhn_ref[...]))
        h_new = (1.0 - z) * n + z * h
        h = jnp.where(t < lens, h_new, h)
    h_ref[...] = h


def _run_gru(words_tensor, lens2, char_table, W_ih, W_hh, b_ih, b_hh):
    f32 = jnp.float32
    wT = W_ih.T
    uT = W_hh.T
    parts = []
    for g in range(3):
        parts.append(wT[:, g * HID:(g + 1) * HID])
    for g in range(3):
        parts.append(uT[:, g * HID:(g + 1) * HID])
    for g in range(3):
        parts.append(b_ih[g * HID:(g + 1) * HID].reshape(1, HID))
    for g in range(3):
        parts.append(b_hh[g * HID:(g + 1) * HID].reshape(1, HID))

    full = lambda shape: pl.BlockSpec(shape, lambda i: (0, 0))
    return pl.pallas_call(
        _gru_block,
        grid=(NW // NB,),
        in_specs=[
            pl.BlockSpec((NB, TW), lambda i: (i, 0)),
            pl.BlockSpec((NB, 1), lambda i: (i, 0)),
            full((CHAR_VOCAB, CHAR_DIM)),
            full((CHAR_DIM, HID)), full((CHAR_DIM, HID)), full((CHAR_DIM, HID)),
            full((HID, HID)), full((HID, HID)), full((HID, HID)),
            full((1, HID)), full((1, HID)), full((1, HID)),
            full((1, HID)), full((1, HID)), full((1, HID)),
        ],
        out_specs=pl.BlockSpec((NB, HID), lambda i: (i, 0)),
        out_shape=jax.ShapeDtypeStruct((NW, HID), f32),
    )(words_tensor, lens2, char_table, *parts)


def _sc_gather(word_table_z, h_words, doc_idx3, chr_idx3):
    mesh = plsc.VectorSubcoreMesh(core_axis_name="c", subcore_axis_name="s")
    info = plsc.get_sparse_core_info()
    nc = info.num_cores

    @functools.partial(
        pl.kernel,
        mesh=mesh,
        out_type=jax.ShapeDtypeStruct((TOT, WORD_DIM + HID), jnp.float32),
        compiler_params=pltpu.CompilerParams(use_tc_tiling_on_sc=False),
        scratch_types=[
            pltpu.VMEM((N_CHUNKS, CHUNK), jnp.int32),
            pltpu.VMEM((N_CHUNKS, CHUNK), jnp.int32),
            pltpu.VMEM((2, CHUNK, WORD_DIM), jnp.float32),
            pltpu.VMEM((2, CHUNK, HID), jnp.float32),
            pltpu.SemaphoreType.DMA,
            pltpu.SemaphoreType.DMA,
        ],
    )
    def k(wt_hbm, h_hbm, didx_hbm, cidx_hbm, out_hbm,
          widx_v, cidx_v, bufw, bufc, semw, semc):
        wid = lax.axis_index("s") * nc + lax.axis_index("c")
        base = wid * PER_W
        pltpu.sync_copy(didx_hbm.at[wid], widx_v)
        pltpu.sync_copy(cidx_hbm.at[wid], cidx_v)

        def gather(j):
            slot = j % 2
            cw = pltpu.async_copy(wt_hbm.at[widx_v.at[j]],
                                  bufw.at[slot], semw)
            cc = pltpu.async_copy(h_hbm.at[cidx_v.at[j]],
                                  bufc.at[slot], semc)
            return cw, cc

        g_cur = gather(0)
        for j in range(N_CHUNKS):
            g_next = gather(j + 1) if j + 1 < N_CHUNKS else None
            g_cur[0].wait()
            g_cur[1].wait()
            slot = j % 2
            row0 = base + j * CHUNK
            pltpu.sync_copy(bufw.at[slot],
                            out_hbm.at[pl.ds(row0, CHUNK), pl.ds(0, WORD_DIM)])
            pltpu.sync_copy(bufc.at[slot],
                            out_hbm.at[pl.ds(row0, CHUNK), pl.ds(WORD_DIM, HID)])
            g_cur = g_next

    return k(word_table_z, h_words, doc_idx3, chr_idx3)


def kernel(words_tensor, words_lengths, doc_tensor, doc_tensor_new_dx,
           char_table, word_table, W_ih, W_hh, b_ih, b_hh):
    lens2 = words_lengths.reshape(NW, 1)
    h = _run_gru(words_tensor, lens2, char_table, W_ih, W_hh, b_ih, b_hh)
    word_table_z = word_table.at[0].set(0.0)
    doc3 = doc_tensor.reshape(N_WORKERS, N_CHUNKS, CHUNK)
    dx3 = doc_tensor_new_dx.reshape(N_WORKERS, N_CHUNKS, CHUNK)
    out = _sc_gather(word_table_z, h, doc3, dx3)
    return out.reshape(B, L, WORD_DIM + HID)


# speedup vs baseline: 5.4504x; 1.3108x over previous
import functools

import jax
import jax.numpy as jnp
from jax import lax
from jax.experimental import pallas as pl
from jax.experimental.pallas import tpu as pltpu
from jax.experimental.pallas import tpu_sc as plsc

CHAR_VOCAB = 128
CHAR_DIM = 16
HID = 64
WORD_DIM = 64
NW = 8192
TW = 16
B = 1024
L = 50
TOT = B * L

NB = 2048

N_WORKERS = 32
PER_W = TOT // N_WORKERS
CHUNKS = [(j * 128, 128) for j in range(12)] + [(1536, 64)]


def _gru_block(chars_ref, lens_ref, ct_ref,
               wxr_ref, wxz_ref, wxn_ref, whr_ref, whz_ref, whn_ref,
               bxr_ref, bxz_ref, bxn_ref, bhr_ref, bhz_ref, bhn_ref,
               h_ref):
    f32 = jnp.float32
    ct = ct_ref[...]
    ct = jnp.where(lax.broadcasted_iota(jnp.int32, ct.shape, 0) == 0, 0.0, ct)
    chars = chars_ref[...]
    lens = lens_ref[...]
    iota_v = lax.broadcasted_iota(jnp.int32, (1, CHAR_VOCAB), 1)
    h = jnp.zeros((chars.shape[0], HID), f32)
    dot = functools.partial(jnp.dot, preferred_element_type=f32)
    for t in range(TW):
        oh = (chars[:, t:t + 1] == iota_v).astype(f32)
        x = dot(oh, ct)
        r = jax.nn.sigmoid(dot(x, wxr_ref[...]) + bxr_ref[...]
                           + dot(h, whr_ref[...]) + bhr_ref[...])
        z = jax.nn.sigmoid(dot(x, wxz_ref[...]) + bxz_ref[...]
                           + dot(h, whz_ref[...]) + bhz_ref[...])
        n = jnp.tanh(dot(x, wxn_ref[...]) + bxn_ref[...]
                     + r * (dot(h, whn_ref[...]) + bhn_ref[...]))
        h_new = (1.0 - z) * n + z * h
        h = jnp.where(t < lens, h_new, h)
    h_ref[...] = h


def _run_gru(words_tensor, lens2, char_table, W_ih, W_hh, b_ih, b_hh):
    f32 = jnp.float32
    wT = W_ih.T
    uT = W_hh.T
    parts = []
    for g in range(3):
        parts.append(wT[:, g * HID:(g + 1) * HID])
    for g in range(3):
        parts.append(uT[:, g * HID:(g + 1) * HID])
    for g in range(3):
        parts.append(b_ih[g * HID:(g + 1) * HID].reshape(1, HID))
    for g in range(3):
        parts.append(b_hh[g * HID:(g + 1) * HID].reshape(1, HID))

    full = lambda shape: pl.BlockSpec(shape, lambda i: (0, 0))
    return pl.pallas_call(
        _gru_block,
        grid=(NW // NB,),
        in_specs=[
            pl.BlockSpec((NB, TW), lambda i: (i, 0)),
            pl.BlockSpec((NB, 1), lambda i: (i, 0)),
            full((CHAR_VOCAB, CHAR_DIM)),
            full((CHAR_DIM, HID)), full((CHAR_DIM, HID)), full((CHAR_DIM, HID)),
            full((HID, HID)), full((HID, HID)), full((HID, HID)),
            full((1, HID)), full((1, HID)), full((1, HID)),
            full((1, HID)), full((1, HID)), full((1, HID)),
        ],
        out_specs=pl.BlockSpec((NB, HID), lambda i: (i, 0)),
        out_shape=jax.ShapeDtypeStruct((NW, HID), f32),
    )(words_tensor, lens2, char_table, *parts)


def _sc_gather(word_table, h_words, doc_idx2, chr_idx2):
    mesh = plsc.VectorSubcoreMesh(core_axis_name="c", subcore_axis_name="s")
    nc = plsc.get_sparse_core_info().num_cores

    @functools.partial(
        pl.kernel,
        mesh=mesh,
        out_type=jax.ShapeDtypeStruct((TOT, WORD_DIM + HID), jnp.float32),
        compiler_params=pltpu.CompilerParams(use_tc_tiling_on_sc=False,
                                             needs_layout_passes=False),
        scratch_types=[
            pltpu.VMEM((PER_W,), jnp.int32),
            pltpu.VMEM((PER_W,), jnp.int32),
            pltpu.VMEM((2, 128, WORD_DIM), jnp.float32),
            pltpu.VMEM((2, 128, HID), jnp.float32),
            pltpu.VMEM((1, WORD_DIM), jnp.float32),
            pltpu.SemaphoreType.DMA,
            pltpu.SemaphoreType.DMA,
        ],
    )
    def k(wt_hbm, h_hbm, didx_hbm, cidx_hbm, out_hbm,
          widx_v, cidx_v, bufw, bufc, zrow_v, semw, semc):
        wid = lax.axis_index("s") * nc + lax.axis_index("c")
        base = wid * PER_W
        pltpu.sync_copy(didx_hbm.at[wid], widx_v)
        pltpu.sync_copy(cidx_hbm.at[wid], cidx_v)
        zeros16f = jnp.zeros((16,), jnp.float32)
        for c in range(WORD_DIM // 16):
            zrow_v[0, pl.ds(c * 16, 16)] = zeros16f

        def gather(j):
            off, n = CHUNKS[j]
            slot = j % 2
            cw = pltpu.async_copy(wt_hbm.at[widx_v.at[pl.ds(off, n)]],
                                  bufw.at[slot, pl.ds(0, n)], semw)
            cc = pltpu.async_copy(h_hbm.at[cidx_v.at[pl.ds(off, n)]],
                                  bufc.at[slot, pl.ds(0, n)], semc)
            return cw, cc

        g_cur = gather(0)
        for j in range(len(CHUNKS)):
            off, n = CHUNKS[j]
            slot = j % 2
            g_next = gather(j + 1) if j + 1 < len(CHUNKS) else None
            g_cur[0].wait()
            g_cur[1].wait()
            row0 = base + off
            pltpu.sync_copy(bufw.at[slot, pl.ds(0, n)],
                            out_hbm.at[pl.ds(row0, n), pl.ds(0, WORD_DIM)])
            pltpu.sync_copy(bufc.at[slot, pl.ds(0, n)],
                            out_hbm.at[pl.ds(row0, n), pl.ds(WORD_DIM, HID)])
            g_cur = g_next

        lane = lax.broadcasted_iota(jnp.int32, (16,), 0)

        def scan_group(g, carry):
            iv = widx_v[pl.ds(g * 16, 16)]
            nzero = plsc.all_reduce_population_count(iv == 0)[0]

            @pl.when(nzero > 0)
            def _():
                def fix_one(r, carry2):
                    hit = plsc.all_reduce_population_count(
                        (lane == r) & (iv == 0))[0]

                    @pl.when(hit > 0)
                    def _():
                        pltpu.sync_copy(
                            zrow_v,
                            out_hbm.at[pl.ds(base + g * 16 + r, 1),
                                       pl.ds(0, WORD_DIM)])
                    return carry2

                lax.fori_loop(0, 16, fix_one, 0)
            return carry

        lax.fori_loop(0, PER_W // 16, scan_group, 0)

    return k(word_table, h_words, doc_idx2, chr_idx2)


def kernel(words_tensor, words_lengths, doc_tensor, doc_tensor_new_dx,
           char_table, word_table, W_ih, W_hh, b_ih, b_hh):
    lens2 = words_lengths.reshape(NW, 1)
    h = _run_gru(words_tensor, lens2, char_table, W_ih, W_hh, b_ih, b_hh)
    doc2 = doc_tensor.T.reshape(N_WORKERS, PER_W)
    dx2 = doc_tensor_new_dx.T.reshape(N_WORKERS, PER_W)
    out = _sc_gather(word_table, h, doc2, dx2)
    return out.reshape(L, B, WORD_DIM + HID).swapaxes(0, 1)
